# baseline (speedup 1.0000x reference)
import functools

import jax
import jax.numpy as jnp
from jax import lax
from jax.experimental import pallas as pl
from jax.experimental.pallas import tpu as pltpu
from jax.experimental.pallas import tpu_sc as plsc

BATCH = 16384
EMBED_DIM = 128
ITEM_FEAT = 256
HIDDEN = 512

NC = 2
NS = 16
NW = NC * NS
B_PER_W = BATCH // NW
IDX_CHUNK = 128
N_CHUNKS = B_PER_W // IDX_CHUNK


def _sc_gather(table, idx2d):
    mesh = plsc.VectorSubcoreMesh(core_axis_name="c", subcore_axis_name="s")

    @functools.partial(
        pl.kernel,
        mesh=mesh,
        out_type=jax.ShapeDtypeStruct((BATCH, EMBED_DIM), jnp.float32),
        scratch_types=[
            pltpu.VMEM((N_CHUNKS, IDX_CHUNK), jnp.int32),
            pltpu.VMEM((B_PER_W, EMBED_DIM), jnp.float32),
            pltpu.SemaphoreType.DMA,
        ],
    )
    def gather_kernel(table_hbm, idx_hbm, out_hbm, idx_v, rows_v, sem):
        wid = lax.axis_index("s") * NC + lax.axis_index("c")
        pltpu.sync_copy(idx_hbm.at[pl.ds(wid * N_CHUNKS, N_CHUNKS)], idx_v)
        for j in range(N_CHUNKS):
            pltpu.async_copy(
                table_hbm.at[idx_v.at[j]],
                rows_v.at[pl.ds(j * IDX_CHUNK, IDX_CHUNK)],
                sem,
            )
        for _ in range(N_CHUNKS):
            pltpu.make_async_copy(
                table_hbm.at[idx_v.at[0]],
                rows_v.at[pl.ds(0, IDX_CHUNK)],
                sem,
            ).wait()
        pltpu.sync_copy(rows_v, out_hbm.at[pl.ds(wid * B_PER_W, B_PER_W)])

    return gather_kernel(table, idx2d)


BLK = 2048
NSTEP = BATCH // BLK


def _mlp_kernel(x_hbm, w1_ref, b1_ref, w2_ref, b2_ref, col_hbm,
                xbuf, colbuf, in_sems, out_sems):
    w1 = w1_ref[...].astype(jnp.bfloat16)
    w2 = w2_ref[...].astype(jnp.bfloat16)
    b1 = b1_ref[...]
    b2 = b2_ref[...]

    def in_copy(j):
        slot = j % 3
        return pltpu.make_async_copy(
            x_hbm.at[pl.ds(j * BLK, BLK), :], xbuf.at[slot], in_sems.at[slot])

    def out_copy(j, slot):
        return pltpu.make_async_copy(
            colbuf.at[slot], col_hbm.at[pl.ds(j * BLK, BLK), :], out_sems.at[slot])

    in_copy(0).start()
    in_copy(1).start()
    in_copy(2).start()
    for j in range(NSTEP):
        slot = j % 2
        in_copy(j).wait()
        x = xbuf[j % 3].astype(jnp.bfloat16)
        h = jnp.dot(x, w1, preferred_element_type=jnp.float32)
        h = jnp.maximum(h + b1, 0.0).astype(jnp.bfloat16)
        col = jnp.dot(h, w2, preferred_element_type=jnp.float32)
        if j >= 2:
            out_copy(j - 2, slot).wait()
        colbuf[slot] = (col + b2).astype(jnp.bfloat16)
        out_copy(j, slot).start()
        if j + 3 < NSTEP:
            in_copy(j + 3).start()
    out_copy(NSTEP - 2, 0).wait()
    out_copy(NSTEP - 1, 1).wait()


def _tc_mlp(x, W1, b1, W2, b2):
    return pl.pallas_call(
        _mlp_kernel,
        in_specs=[
            pl.BlockSpec(memory_space=pl.ANY),
            pl.BlockSpec((ITEM_FEAT, HIDDEN), lambda: (0, 0)),
            pl.BlockSpec((1, HIDDEN), lambda: (0, 0)),
            pl.BlockSpec((HIDDEN, EMBED_DIM), lambda: (0, 0)),
            pl.BlockSpec((1, EMBED_DIM), lambda: (0, 0)),
        ],
        out_specs=pl.BlockSpec(memory_space=pl.ANY),
        out_shape=jax.ShapeDtypeStruct((BATCH, EMBED_DIM), jnp.bfloat16),
        scratch_shapes=[
            pltpu.VMEM((3, BLK, ITEM_FEAT), jnp.float32),
            pltpu.VMEM((2, BLK, EMBED_DIM), jnp.bfloat16),
            pltpu.SemaphoreType.DMA((3,)),
            pltpu.SemaphoreType.DMA((2,)),
        ],
    )(x, W1, b1, W2, b2)


DOT_BLK = 4096


def _dot_kernel(rows_ref, col_ref, out_ref):
    prod = rows_ref[...] * col_ref[...].astype(jnp.float32)
    ones = jnp.ones((1, EMBED_DIM), jnp.float32)
    s = jax.lax.dot_general(
        ones,
        prod,
        ((( 1,), (1,)), ((), ())),
        preferred_element_type=jnp.float32,
    )
    out_ref[...] = s.reshape(1, 1, DOT_BLK)


def _tc_dot(rows, col):
    grid = (BATCH // DOT_BLK,)
    out = pl.pallas_call(
        _dot_kernel,
        grid=grid,
        in_specs=[
            pl.BlockSpec((DOT_BLK, EMBED_DIM), lambda i: (i, 0)),
            pl.BlockSpec((DOT_BLK, EMBED_DIM), lambda i: (i, 0)),
        ],
        out_specs=pl.BlockSpec((1, 1, DOT_BLK), lambda i: (i, 0, 0)),
        out_shape=jax.ShapeDtypeStruct((BATCH // DOT_BLK, 1, DOT_BLK), jnp.float32),
    )(rows, col)
    return out.reshape(BATCH)


def kernel(uid, item_features, row_embeddings, W1, b1, W2, b2):
    idx2d = uid.astype(jnp.int32).reshape(BATCH // IDX_CHUNK, IDX_CHUNK)
    rows = _sc_gather(row_embeddings, idx2d)
    col = _tc_mlp(
        item_features,
        W1,
        b1.reshape(1, HIDDEN),
        W2,
        b2.reshape(1, EMBED_DIM),
    )
    return _tc_dot(rows, col)


# speedup vs baseline: 1.0798x; 1.0798x over previous
import functools

import jax
import jax.numpy as jnp
from jax import lax
from jax.experimental import pallas as pl
from jax.experimental.pallas import tpu as pltpu
from jax.experimental.pallas import tpu_sc as plsc

BATCH = 16384
EMBED_DIM = 128
ITEM_FEAT = 256
HIDDEN = 512

NC = 2
NS = 16
NW = NC * NS
B_PER_W = BATCH // NW
IDX_CHUNK = 128
N_CHUNKS = B_PER_W // IDX_CHUNK


def _sc_gather(table, idx2d):
    mesh = plsc.VectorSubcoreMesh(core_axis_name="c", subcore_axis_name="s")

    @functools.partial(
        pl.kernel,
        mesh=mesh,
        out_type=jax.ShapeDtypeStruct((BATCH, EMBED_DIM), jnp.float32),
        scratch_types=[
            pltpu.VMEM((N_CHUNKS, IDX_CHUNK), jnp.int32),
            pltpu.VMEM((B_PER_W, EMBED_DIM), jnp.float32),
            pltpu.SemaphoreType.DMA,
        ],
    )
    def gather_kernel(table_hbm, idx_hbm, out_hbm, idx_v, rows_v, sem):
        wid = lax.axis_index("s") * NC + lax.axis_index("c")
        pltpu.sync_copy(idx_hbm.at[pl.ds(wid * N_CHUNKS, N_CHUNKS)], idx_v)
        for j in range(N_CHUNKS):
            pltpu.async_copy(
                table_hbm.at[idx_v.at[j]],
                rows_v.at[pl.ds(j * IDX_CHUNK, IDX_CHUNK)],
                sem,
            )
        for _ in range(N_CHUNKS):
            pltpu.make_async_copy(
                table_hbm.at[idx_v.at[0]],
                rows_v.at[pl.ds(0, IDX_CHUNK)],
                sem,
            ).wait()
        pltpu.sync_copy(rows_v, out_hbm.at[pl.ds(wid * B_PER_W, B_PER_W)])

    return gather_kernel(table, idx2d)


BLK = 4096


def _mlp_kernel(x_ref, w1_ref, b1_ref, w2_ref, b2_ref, col_ref):
    x = x_ref[...].astype(jnp.bfloat16)
    w1 = w1_ref[...].astype(jnp.bfloat16)
    h = jnp.dot(x, w1, preferred_element_type=jnp.float32)
    h = jnp.maximum(h + b1_ref[...], 0.0).astype(jnp.bfloat16)
    w2 = w2_ref[...].astype(jnp.bfloat16)
    col = jnp.dot(h, w2, preferred_element_type=jnp.float32)
    col_ref[...] = (col + b2_ref[...]).astype(jnp.bfloat16)


def _tc_mlp(x, W1, b1, W2, b2):
    grid = (BATCH // BLK,)
    return pl.pallas_call(
        _mlp_kernel,
        grid=grid,
        in_specs=[
            pl.BlockSpec((BLK, ITEM_FEAT), lambda i: (i, 0)),
            pl.BlockSpec((ITEM_FEAT, HIDDEN), lambda i: (0, 0)),
            pl.BlockSpec((1, HIDDEN), lambda i: (0, 0)),
            pl.BlockSpec((HIDDEN, EMBED_DIM), lambda i: (0, 0)),
            pl.BlockSpec((1, EMBED_DIM), lambda i: (0, 0)),
        ],
        out_specs=pl.BlockSpec((BLK, EMBED_DIM), lambda i: (i, 0)),
        out_shape=jax.ShapeDtypeStruct((BATCH, EMBED_DIM), jnp.bfloat16),
    )(x, W1, b1, W2, b2)


DOT_BLK = 2048


def _dot_kernel(rows_ref, col_ref, out_ref):
    prod = rows_ref[...] * col_ref[...].astype(jnp.float32)
    ones = jnp.ones((1, EMBED_DIM), jnp.float32)
    s = jax.lax.dot_general(
        ones,
        prod,
        ((( 1,), (1,)), ((), ())),
        preferred_element_type=jnp.float32,
    )
    out_ref[...] = s.reshape(1, 1, DOT_BLK)


def _tc_dot(rows, col):
    grid = (BATCH // DOT_BLK,)
    out = pl.pallas_call(
        _dot_kernel,
        grid=grid,
        in_specs=[
            pl.BlockSpec((DOT_BLK, EMBED_DIM), lambda i: (i, 0)),
            pl.BlockSpec((DOT_BLK, EMBED_DIM), lambda i: (i, 0)),
        ],
        out_specs=pl.BlockSpec((1, 1, DOT_BLK), lambda i: (i, 0, 0)),
        out_shape=jax.ShapeDtypeStruct((BATCH // DOT_BLK, 1, DOT_BLK), jnp.float32),
    )(rows, col)
    return out.reshape(BATCH)


def kernel(uid, item_features, row_embeddings, W1, b1, W2, b2):
    idx2d = uid.astype(jnp.int32).reshape(BATCH // IDX_CHUNK, IDX_CHUNK)
    rows = _sc_gather(row_embeddings, idx2d)
    col = _tc_mlp(
        item_features,
        W1,
        b1.reshape(1, HIDDEN),
        W2,
        b2.reshape(1, EMBED_DIM),
    )
    return _tc_dot(rows, col)


# speedup vs baseline: 1.1004x; 1.0191x over previous
import functools

import jax
import jax.numpy as jnp
from jax import lax
from jax.experimental import pallas as pl
from jax.experimental.pallas import tpu as pltpu
from jax.experimental.pallas import tpu_sc as plsc

BATCH = 16384
EMBED_DIM = 128
ITEM_FEAT = 256
HIDDEN = 512

NC = 2
NS = 16
NW = NC * NS
B_PER_W = BATCH // NW
IDX_CHUNK = 128
N_CHUNKS = B_PER_W // IDX_CHUNK


def _sc_gather(table, idx2d):
    mesh = plsc.VectorSubcoreMesh(core_axis_name="c", subcore_axis_name="s")

    @functools.partial(
        pl.kernel,
        mesh=mesh,
        out_type=jax.ShapeDtypeStruct((BATCH, EMBED_DIM), jnp.float32),
        scratch_types=[
            pltpu.VMEM((N_CHUNKS, IDX_CHUNK), jnp.int32),
            pltpu.VMEM((B_PER_W, EMBED_DIM), jnp.float32),
            pltpu.SemaphoreType.DMA,
        ],
    )
    def gather_kernel(table_hbm, idx_hbm, out_hbm, idx_v, rows_v, sem):
        wid = lax.axis_index("s") * NC + lax.axis_index("c")
        pltpu.sync_copy(idx_hbm.at[pl.ds(wid * N_CHUNKS, N_CHUNKS)], idx_v)
        for j in range(N_CHUNKS):
            pltpu.async_copy(
                table_hbm.at[idx_v.at[j]],
                rows_v.at[pl.ds(j * IDX_CHUNK, IDX_CHUNK)],
                sem,
            )
        for _ in range(N_CHUNKS):
            pltpu.make_async_copy(
                table_hbm.at[idx_v.at[0]],
                rows_v.at[pl.ds(0, IDX_CHUNK)],
                sem,
            ).wait()
        pltpu.sync_copy(rows_v, out_hbm.at[pl.ds(wid * B_PER_W, B_PER_W)])

    return gather_kernel(table, idx2d)


BLK = 2048


def _mlp_kernel(x_ref, w1_ref, b1_ref, w2_ref, b2_ref, col_ref):
    x = x_ref[...].astype(jnp.bfloat16)
    w1 = w1_ref[...].astype(jnp.bfloat16)
    h = jnp.dot(x, w1, preferred_element_type=jnp.float32).astype(jnp.bfloat16)
    b1 = b1_ref[...].astype(jnp.bfloat16)
    h = jnp.maximum(h + b1, jnp.bfloat16(0.0))
    w2 = w2_ref[...].astype(jnp.bfloat16)
    col = jnp.dot(h, w2, preferred_element_type=jnp.float32)
    col_ref[...] = (col + b2_ref[...]).astype(jnp.bfloat16)


def _tc_mlp(x, W1, b1, W2, b2):
    grid = (BATCH // BLK,)
    return pl.pallas_call(
        _mlp_kernel,
        grid=grid,
        in_specs=[
            pl.BlockSpec((BLK, ITEM_FEAT), lambda i: (i, 0)),
            pl.BlockSpec((ITEM_FEAT, HIDDEN), lambda i: (0, 0)),
            pl.BlockSpec((1, HIDDEN), lambda i: (0, 0)),
            pl.BlockSpec((HIDDEN, EMBED_DIM), lambda i: (0, 0)),
            pl.BlockSpec((1, EMBED_DIM), lambda i: (0, 0)),
        ],
        out_specs=pl.BlockSpec((BLK, EMBED_DIM), lambda i: (i, 0)),
        out_shape=jax.ShapeDtypeStruct((BATCH, EMBED_DIM), jnp.bfloat16),
    )(x, W1, b1, W2, b2)


DOT_BLK = 4096


def _dot_kernel(rows_ref, col_ref, out_ref):
    prod = rows_ref[...] * col_ref[...].astype(jnp.float32)
    ones = jnp.ones((1, EMBED_DIM), jnp.float32)
    s = jax.lax.dot_general(
        ones,
        prod,
        ((( 1,), (1,)), ((), ())),
        preferred_element_type=jnp.float32,
    )
    out_ref[...] = s.reshape(1, 1, DOT_BLK)


def _tc_dot(rows, col):
    grid = (BATCH // DOT_BLK,)
    out = pl.pallas_call(
        _dot_kernel,
        grid=grid,
        in_specs=[
            pl.BlockSpec((DOT_BLK, EMBED_DIM), lambda i: (i, 0)),
            pl.BlockSpec((DOT_BLK, EMBED_DIM), lambda i: (i, 0)),
        ],
        out_specs=pl.BlockSpec((1, 1, DOT_BLK), lambda i: (i, 0, 0)),
        out_shape=jax.ShapeDtypeStruct((BATCH // DOT_BLK, 1, DOT_BLK), jnp.float32),
    )(rows, col)
    return out.reshape(BATCH)


def kernel(uid, item_features, row_embeddings, W1, b1, W2, b2):
    idx2d = uid.astype(jnp.int32).reshape(BATCH // IDX_CHUNK, IDX_CHUNK)
    rows = _sc_gather(row_embeddings, idx2d)
    col = _tc_mlp(
        item_features,
        W1,
        b1.reshape(1, HIDDEN),
        W2,
        b2.reshape(1, EMBED_DIM),
    )
    return _tc_dot(rows, col)
